# baseline (speedup 1.0000x reference)
import functools

import jax
import jax.numpy as jnp
from jax import lax
from jax.experimental import pallas as pl
from jax.experimental.pallas import tpu as pltpu
from jax.experimental.pallas import tpu_sc as plsc

NUM_CORES = 2
NUM_SUBCORES = 16
NUM_WORKERS = NUM_CORES * NUM_SUBCORES
CHUNK = 128
NBUF = 2


@jax.jit
def kernel(inputs, embedding):
    batch, hist = inputs.shape
    vocab, feat = embedding.shape
    total = batch * hist
    rows_per_worker = total // NUM_WORKERS
    nchunk = rows_per_worker // CHUNK

    idx3d = inputs.T.astype(jnp.int32).reshape(NUM_WORKERS, nchunk, CHUNK)

    mesh = plsc.VectorSubcoreMesh(
        core_axis_name="c",
        subcore_axis_name="s",
        num_cores=NUM_CORES,
        num_subcores=NUM_SUBCORES,
    )

    @functools.partial(
        pl.kernel,
        mesh=mesh,
        out_type=jax.ShapeDtypeStruct((total, feat), jnp.float32),
        scratch_types=[
            pltpu.VMEM((nchunk, CHUNK), jnp.int32),
            [pltpu.VMEM((CHUNK, feat), jnp.float32) for _ in range(NBUF)],
            [pltpu.SemaphoreType.DMA for _ in range(NBUF)],
        ],
    )
    def gather_kernel(idx_hbm, table_hbm, out_hbm, idx_v, bufs, sem_g):
        wid = lax.axis_index("s") * NUM_CORES + lax.axis_index("c")
        cbase = wid * nchunk

        pltpu.sync_copy(idx_hbm.at[wid], idx_v)

        def fire_gather(j, b):
            pltpu.async_copy(table_hbm.at[idx_v.at[j]], bufs[b], sem_g[b])

        def wait_gather(j, b):
            pltpu.make_async_copy(
                table_hbm.at[idx_v.at[j]], bufs[b], sem_g[b]
            ).wait()

        for b in range(NBUF):
            fire_gather(b, b)

        @pl.loop(0, nchunk, step=NBUF)
        def _(g):
            for b in range(NBUF):
                j = g + b
                wait_gather(j, b)
                pltpu.sync_copy(
                    bufs[b], out_hbm.at[pl.ds((cbase + j) * CHUNK, CHUNK)]
                )

                @pl.when(j + NBUF < nchunk)
                def _fire():
                    fire_gather(j + NBUF, b)

    out = gather_kernel(idx3d, embedding)
    return out.reshape(hist, batch, feat).transpose(1, 0, 2)


# speedup vs baseline: 1.0162x; 1.0162x over previous
import functools

import jax
import jax.numpy as jnp
from jax import lax
from jax.experimental import pallas as pl
from jax.experimental.pallas import tpu as pltpu
from jax.experimental.pallas import tpu_sc as plsc

NUM_CORES = 2
NUM_SUBCORES = 16
NUM_WORKERS = NUM_CORES * NUM_SUBCORES
CHUNK = 128
NBUF = 5
AHEAD = 3


@jax.jit
def kernel(inputs, embedding):
    batch, hist = inputs.shape
    vocab, feat = embedding.shape
    total = batch * hist
    rows_per_worker = total // NUM_WORKERS
    nchunk = rows_per_worker // CHUNK

    idx_flat = inputs.T.astype(jnp.int32).reshape(total)

    mesh = plsc.VectorSubcoreMesh(
        core_axis_name="c",
        subcore_axis_name="s",
        num_cores=NUM_CORES,
        num_subcores=NUM_SUBCORES,
    )

    @functools.partial(
        pl.kernel,
        mesh=mesh,
        out_type=jax.ShapeDtypeStruct((total, feat), jnp.float32),
        scratch_types=[
            pltpu.VMEM((rows_per_worker,), jnp.int32),
            [pltpu.VMEM((CHUNK, feat), jnp.float32) for _ in range(NBUF)],
            [pltpu.SemaphoreType.DMA for _ in range(NBUF)],
            [pltpu.SemaphoreType.DMA for _ in range(NBUF)],
        ],
    )
    def gather_kernel(idx_hbm, table_hbm, out_hbm, idx_v, bufs, sem_g, sem_s):
        wid = lax.axis_index("s") * NUM_CORES + lax.axis_index("c")
        cbase = wid * nchunk

        pltpu.sync_copy(
            idx_hbm.at[pl.ds(cbase * CHUNK, rows_per_worker)], idx_v
        )

        def fire_gather(j, b):
            pltpu.async_copy(
                table_hbm.at[idx_v.at[pl.ds(j * CHUNK, CHUNK)]], bufs[b], sem_g[b]
            )

        def wait_gather(j, b):
            pltpu.make_async_copy(
                table_hbm.at[idx_v.at[pl.ds(j * CHUNK, CHUNK)]], bufs[b], sem_g[b]
            ).wait()

        def fire_scatter(j, b):
            pltpu.async_copy(
                bufs[b], out_hbm.at[pl.ds((cbase + j) * CHUNK, CHUNK)], sem_s[b]
            )

        def wait_scatter(j, b):
            pltpu.make_async_copy(
                bufs[b], out_hbm.at[pl.ds((cbase + j) * CHUNK, CHUNK)], sem_s[b]
            ).wait()

        for b in range(AHEAD):
            fire_gather(b, b)

        @pl.loop(0, nchunk, step=NBUF)
        def _(g):
            for b in range(NBUF):
                j = g + b
                wait_gather(j, b)
                fire_scatter(j, b)
                f = j + AHEAD
                bf = (b + AHEAD) % NBUF

                @pl.when(f < nchunk)
                def _fire():
                    @pl.when(f >= NBUF)
                    def _drain():
                        wait_scatter(f - NBUF, bf)

                    fire_gather(f, bf)

        for jt in range(nchunk - NBUF, nchunk):
            wait_scatter(jt, jt % NBUF)

    out = gather_kernel(idx_flat, embedding)
    return out.reshape(hist, batch, feat).transpose(1, 0, 2)


# speedup vs baseline: 1.0173x; 1.0011x over previous
import functools

import jax
import jax.numpy as jnp
from jax import lax
from jax.experimental import pallas as pl
from jax.experimental.pallas import tpu as pltpu
from jax.experimental.pallas import tpu_sc as plsc

NUM_CORES = 2
NUM_SUBCORES = 16
NUM_WORKERS = NUM_CORES * NUM_SUBCORES
CHUNK = 128
NBUF = 5
AHEAD = 4


@jax.jit
def kernel(inputs, embedding):
    batch, hist = inputs.shape
    vocab, feat = embedding.shape
    total = batch * hist
    rows_per_worker = total // NUM_WORKERS
    nchunk = rows_per_worker // CHUNK

    idx3d = inputs.T.astype(jnp.int32).reshape(NUM_WORKERS, nchunk, CHUNK)

    mesh = plsc.VectorSubcoreMesh(
        core_axis_name="c",
        subcore_axis_name="s",
        num_cores=NUM_CORES,
        num_subcores=NUM_SUBCORES,
    )

    @functools.partial(
        pl.kernel,
        mesh=mesh,
        out_type=jax.ShapeDtypeStruct((total, feat), jnp.float32),
        scratch_types=[
            pltpu.VMEM((nchunk, CHUNK), jnp.int32),
            [pltpu.VMEM((CHUNK, feat), jnp.float32) for _ in range(NBUF)],
            [pltpu.SemaphoreType.DMA for _ in range(NBUF)],
            [pltpu.SemaphoreType.DMA for _ in range(NBUF)],
        ],
    )
    def gather_kernel(idx_hbm, table_hbm, out_hbm, idx_v, bufs, sem_g, sem_s):
        wid = lax.axis_index("s") * NUM_CORES + lax.axis_index("c")
        cbase = wid * nchunk

        pltpu.sync_copy(idx_hbm.at[wid], idx_v)

        def fire_gather(j, b):
            pltpu.async_copy(table_hbm.at[idx_v.at[j]], bufs[b], sem_g[b])

        def wait_gather(j, b):
            pltpu.make_async_copy(
                table_hbm.at[idx_v.at[j]], bufs[b], sem_g[b]
            ).wait()

        def fire_scatter(j, b):
            pltpu.async_copy(
                bufs[b], out_hbm.at[pl.ds((cbase + j) * CHUNK, CHUNK)], sem_s[b]
            )

        def wait_scatter(j, b):
            pltpu.make_async_copy(
                bufs[b], out_hbm.at[pl.ds((cbase + j) * CHUNK, CHUNK)], sem_s[b]
            ).wait()

        for b in range(AHEAD):
            fire_gather(b, b)

        @pl.loop(0, nchunk, step=NBUF)
        def _(g):
            for b in range(NBUF):
                j = g + b
                wait_gather(j, b)
                fire_scatter(j, b)
                f = j + AHEAD
                bf = (b + AHEAD) % NBUF

                @pl.when(f < nchunk)
                def _fire():
                    @pl.when(f >= NBUF)
                    def _drain():
                        wait_scatter(f - NBUF, bf)

                    fire_gather(f, bf)

        for jt in range(nchunk - NBUF, nchunk):
            wait_scatter(jt, jt % NBUF)

    out = gather_kernel(idx3d, embedding)
    return out.reshape(hist, batch, feat).transpose(1, 0, 2)


# speedup vs baseline: 1.0190x; 1.0017x over previous
import functools

import jax
import jax.numpy as jnp
from jax import lax
from jax.experimental import pallas as pl
from jax.experimental.pallas import tpu as pltpu
from jax.experimental.pallas import tpu_sc as plsc

NUM_CORES = 2
NUM_SUBCORES = 16
NUM_WORKERS = NUM_CORES * NUM_SUBCORES
CHUNK = 128
NBUF = 5
AHEAD = 3


@jax.jit
def kernel(inputs, embedding):
    batch, hist = inputs.shape
    vocab, feat = embedding.shape
    total = batch * hist
    rows_per_worker = total // NUM_WORKERS
    nchunk = rows_per_worker // CHUNK

    idx3d = inputs.T.astype(jnp.int32).reshape(NUM_WORKERS, nchunk, CHUNK)

    mesh = plsc.VectorSubcoreMesh(
        core_axis_name="c",
        subcore_axis_name="s",
        num_cores=NUM_CORES,
        num_subcores=NUM_SUBCORES,
    )

    @functools.partial(
        pl.kernel,
        mesh=mesh,
        out_type=jax.ShapeDtypeStruct((total, feat), jnp.float32),
        scratch_types=[
            pltpu.VMEM((nchunk, CHUNK), jnp.int32),
            [pltpu.VMEM((CHUNK, feat), jnp.float32) for _ in range(NBUF)],
            [pltpu.SemaphoreType.DMA for _ in range(NBUF)],
            [pltpu.SemaphoreType.DMA for _ in range(NBUF)],
        ],
    )
    def gather_kernel(idx_hbm, table_hbm, out_hbm, idx_v, bufs, sem_g, sem_s):
        wid = lax.axis_index("s") * NUM_CORES + lax.axis_index("c")
        cbase = wid * nchunk

        pltpu.sync_copy(idx_hbm.at[wid], idx_v)

        def fire_gather(j, b):
            pltpu.async_copy(table_hbm.at[idx_v.at[j]], bufs[b], sem_g[b])

        def wait_gather(j, b):
            pltpu.make_async_copy(
                table_hbm.at[idx_v.at[j]], bufs[b], sem_g[b]
            ).wait()

        def fire_scatter(j, b):
            pltpu.async_copy(
                bufs[b], out_hbm.at[pl.ds((cbase + j) * CHUNK, CHUNK)], sem_s[b]
            )

        def wait_scatter(j, b):
            pltpu.make_async_copy(
                bufs[b], out_hbm.at[pl.ds((cbase + j) * CHUNK, CHUNK)], sem_s[b]
            ).wait()

        for b in range(AHEAD):
            fire_gather(b, b)

        @pl.loop(0, nchunk, step=NBUF)
        def _(g):
            for b in range(NBUF):
                j = g + b
                wait_gather(j, b)
                fire_scatter(j, b)
                f = j + AHEAD
                bf = (b + AHEAD) % NBUF

                @pl.when(f < nchunk)
                def _fire():
                    @pl.when(f >= NBUF)
                    def _drain():
                        wait_scatter(f - NBUF, bf)

                    fire_gather(f, bf)

        for jt in range(nchunk - NBUF, nchunk):
            wait_scatter(jt, jt % NBUF)

    out = gather_kernel(idx3d, embedding)
    return out.reshape(hist, batch, feat).transpose(1, 0, 2)
